# baseline (speedup 1.0000x reference)
import functools

import jax
import jax.numpy as jnp
from jax import lax
from jax.experimental import pallas as pl
from jax.experimental.pallas import tpu as pltpu
from jax.experimental.pallas import tpu_sc as plsc

_LANES = 16
_PAD = 128
_TCOLS = 16384


def _transpose_block(x_ref, y_ref, ox_ref, oy_ref):
    ox_ref[...] = x_ref[...].T
    oy_ref[...] = y_ref[...].T


@functools.cache
def _build_tc_transpose(D, V):
    grid = -(-V // _TCOLS)
    return pl.pallas_call(
        _transpose_block,
        grid=(grid,),
        in_specs=[
            pl.BlockSpec((D, _TCOLS), lambda c: (0, c)),
            pl.BlockSpec((D, _TCOLS), lambda c: (0, c)),
        ],
        out_specs=[
            pl.BlockSpec((_TCOLS, D), lambda c: (c, 0)),
            pl.BlockSpec((_TCOLS, D), lambda c: (c, 0)),
        ],
        out_shape=(
            jax.ShapeDtypeStruct((V, D), jnp.float32),
            jax.ShapeDtypeStruct((V, D), jnp.float32),
        ),
    )


@functools.cache
def _build_sc_gather(B, D, NC, NS):
    NW = NC * NS
    b_per_w = B // NW
    mesh = plsc.VectorSubcoreMesh(core_axis_name="c", subcore_axis_name="s")

    @functools.partial(
        pl.kernel,
        mesh=mesh,
        out_type=(
            jax.ShapeDtypeStruct((B,), jnp.float32),
            jax.ShapeDtypeStruct((B, _PAD), jnp.float32),
            jax.ShapeDtypeStruct((B, _PAD), jnp.float32),
        ),
        scratch_types=[
            pltpu.VMEM((b_per_w,), jnp.int32),
            pltpu.VMEM((b_per_w,), jnp.int32),
            pltpu.VMEM((b_per_w // 2, _PAD), jnp.float32),
            pltpu.VMEM((b_per_w // 2, _PAD), jnp.float32),
            pltpu.VMEM((b_per_w,), jnp.float32),
            pltpu.SemaphoreType.DMA,
            pltpu.SemaphoreType.DMA,
        ],
        compiler_params=pltpu.CompilerParams(
            needs_layout_passes=False, use_tc_tiling_on_sc=True),
    )
    def run(user_h, item_h, gu_h, gi_h, xui_h, gu_out_h, gi_out_h,
            uidx_v, iidx_v, urows_v, irows_v, xui_v, gsem, osem):
        wid = lax.axis_index("s") * NC + lax.axis_index("c")
        base = wid * b_per_w

        half = b_per_w // 2
        quarter = half // 2
        pltpu.sync_copy(user_h.at[pl.ds(base, b_per_w)], uidx_v)
        pltpu.sync_copy(item_h.at[pl.ds(base, b_per_w)], iidx_v)

        lane = lax.iota(jnp.int32, _LANES)

        for h in range(2):
            hbase = h * half

            def fetch(g, carry):
                uv = uidx_v[pl.ds(hbase + g * _LANES, _LANES)]
                iv = iidx_v[pl.ds(hbase + g * _LANES, _LANES)]
                for l in range(_LANES):
                    rr = g * _LANES + l
                    pltpu.async_copy(
                        gu_h.at[uv[l]], urows_v.at[rr, pl.ds(0, D)], gsem)
                    pltpu.async_copy(
                        gi_h.at[iv[l]], irows_v.at[rr, pl.ds(0, D)], gsem)
                return carry

            lax.fori_loop(0, half // _LANES, fetch, 0)

            pltpu.make_async_copy(
                gu_out_h.at[pl.ds(0, quarter)], urows_v.at[pl.ds(0, quarter)],
                gsem).wait()
            pltpu.make_async_copy(
                gi_out_h.at[pl.ds(0, quarter)], irows_v.at[pl.ds(0, quarter)],
                gsem).wait()

            wu = pltpu.async_copy(
                urows_v, gu_out_h.at[pl.ds(base + hbase, half)], osem)
            wi = pltpu.async_copy(
                irows_v, gi_out_h.at[pl.ds(base + hbase, half)], osem)

            def group(g, carry):
                vec = jnp.zeros((_LANES,), jnp.float32)
                row = g * _LANES + lane
                for c in range(D):
                    col = jnp.full((_LANES,), c, jnp.int32)
                    uv = plsc.load_gather(urows_v, [row, col])
                    iv = plsc.load_gather(irows_v, [row, col])
                    vec = vec + uv * iv
                xui_v[pl.ds(hbase + g * _LANES, _LANES)] = vec
                return carry

            lax.fori_loop(0, half // _LANES, group, 0)
            wu.wait()
            wi.wait()

        pltpu.sync_copy(xui_v, xui_h.at[pl.ds(base, b_per_w)])

    return run


def kernel(user, item, Gu, Gi):
    B = user.shape[0]
    V, D = Gu.shape
    info = plsc.get_sparse_core_info()
    NC, NS = info.num_cores, info.num_subcores
    gu_lin, gi_lin = _build_tc_transpose(D, V)(Gu.T, Gi.T)
    run = _build_sc_gather(B, D, NC, NS)
    xui, gu_pad, gi_pad = run(user, item, gu_lin, gi_lin)
    return (xui, gu_pad[:, :D], gi_pad[:, :D])


# speedup vs baseline: 1.0545x; 1.0545x over previous
import functools

import jax
import jax.numpy as jnp
from jax import lax
from jax.experimental import pallas as pl
from jax.experimental.pallas import tpu as pltpu
from jax.experimental.pallas import tpu_sc as plsc

_LANES = 16
_PAD = 128
_TCOLS = 16384


def _transpose_block(x_ref, y_ref, ox_ref, oy_ref):
    ox_ref[...] = x_ref[...].T
    oy_ref[...] = y_ref[...].T


@functools.cache
def _build_tc_transpose(D, V):
    grid = -(-V // _TCOLS)
    return pl.pallas_call(
        _transpose_block,
        grid=(grid,),
        in_specs=[
            pl.BlockSpec((D, _TCOLS), lambda c: (0, c)),
            pl.BlockSpec((D, _TCOLS), lambda c: (0, c)),
        ],
        out_specs=[
            pl.BlockSpec((_TCOLS, D), lambda c: (c, 0)),
            pl.BlockSpec((_TCOLS, D), lambda c: (c, 0)),
        ],
        out_shape=(
            jax.ShapeDtypeStruct((V, D), jnp.float32),
            jax.ShapeDtypeStruct((V, D), jnp.float32),
        ),
    )


@functools.cache
def _build_sc_gather(B, D, NC, NS):
    NW = NC * NS
    b_per_w = B // NW
    mesh = plsc.VectorSubcoreMesh(core_axis_name="c", subcore_axis_name="s")

    @functools.partial(
        pl.kernel,
        mesh=mesh,
        out_type=(
            jax.ShapeDtypeStruct((B,), jnp.float32),
            jax.ShapeDtypeStruct((B, _PAD), jnp.float32),
            jax.ShapeDtypeStruct((B, _PAD), jnp.float32),
        ),
        scratch_types=[
            pltpu.VMEM((b_per_w,), jnp.int32),
            pltpu.VMEM((b_per_w,), jnp.int32),
            pltpu.VMEM((b_per_w // 2, _PAD), jnp.float32),
            pltpu.VMEM((b_per_w // 2, _PAD), jnp.float32),
            pltpu.VMEM((b_per_w,), jnp.float32),
            pltpu.SemaphoreType.DMA,
            pltpu.SemaphoreType.DMA,
        ],
        compiler_params=pltpu.CompilerParams(
            needs_layout_passes=False, use_tc_tiling_on_sc=True),
    )
    def run(user_h, item_h, gu_h, gi_h, xui_h, gu_out_h, gi_out_h,
            uidx_v, iidx_v, urows_v, irows_v, xui_v, gsem, osem):
        wid = lax.axis_index("s") * NC + lax.axis_index("c")
        base = wid * b_per_w

        half = b_per_w // 2
        quarter = half // 2
        pltpu.sync_copy(user_h.at[pl.ds(base, b_per_w)], uidx_v)
        pltpu.sync_copy(item_h.at[pl.ds(base, b_per_w)], iidx_v)

        lane = lax.iota(jnp.int32, _LANES)

        for h in range(2):
            hbase = h * half

            def fetch(g, carry):
                uv = uidx_v[pl.ds(hbase + g * _LANES, _LANES)]
                iv = iidx_v[pl.ds(hbase + g * _LANES, _LANES)]
                for l in range(_LANES):
                    rr = g * _LANES + l
                    pltpu.async_copy(
                        gu_h.at[uv[l]], urows_v.at[rr, pl.ds(0, D)], gsem)
                    pltpu.async_copy(
                        gi_h.at[iv[l]], irows_v.at[rr, pl.ds(0, D)], gsem)
                return carry

            lax.fori_loop(0, half // _LANES, fetch, 0)

            pltpu.make_async_copy(
                gu_out_h.at[pl.ds(0, quarter)], urows_v.at[pl.ds(0, quarter)],
                gsem).wait()
            pltpu.make_async_copy(
                gi_out_h.at[pl.ds(0, quarter)], irows_v.at[pl.ds(0, quarter)],
                gsem).wait()

            wu = pltpu.async_copy(
                urows_v, gu_out_h.at[pl.ds(base + hbase, half)], osem)
            wi = pltpu.async_copy(
                irows_v, gi_out_h.at[pl.ds(base + hbase, half)], osem)

            def group(g, carry):
                vec = jnp.zeros((_LANES,), jnp.float32)
                for l in range(_LANES):
                    r = g * _LANES + l
                    acc = jnp.zeros((_LANES,), jnp.float32)
                    for c in range(0, D, _LANES):
                        acc = acc + (urows_v[r, pl.ds(c, _LANES)]
                                     * irows_v[r, pl.ds(c, _LANES)])
                    vec = jnp.where(lane == l, jnp.sum(acc), vec)
                xui_v[pl.ds(hbase + g * _LANES, _LANES)] = vec
                return carry

            lax.fori_loop(0, half // _LANES, group, 0)
            wu.wait()
            wi.wait()

        pltpu.sync_copy(xui_v, xui_h.at[pl.ds(base, b_per_w)])

    return run


def kernel(user, item, Gu, Gi):
    B = user.shape[0]
    V, D = Gu.shape
    info = plsc.get_sparse_core_info()
    NC, NS = info.num_cores, info.num_subcores
    gu_lin, gi_lin = _build_tc_transpose(D, V)(Gu.T, Gi.T)
    run = _build_sc_gather(B, D, NC, NS)
    xui, gu_pad, gi_pad = run(user, item, gu_lin, gi_lin)
    return (xui, gu_pad[:, :D], gi_pad[:, :D])
